# baseline (device time: 90573 ns/iter reference)
import jax
import jax.numpy as jnp
from jax import lax
from jax.experimental import pallas as pl
from jax.experimental.pallas import tpu as pltpu

N_DEV = 8
N_SUB = 4


def kernel(x, w_mat):
    m_global, k_shard = x.shape
    _, n = w_mat.shape
    m_per = m_global // N_DEV
    nh = n // 2
    ns = nh // N_SUB

    def body(x_ref, w_ref, out_ref,
             send_r, send_l, recv_r, recv_l,
             send_sems_r, send_sems_l, recv_sems_r, recv_sems_l):
        my = lax.axis_index("i")
        left = lax.rem(my + N_DEV - 1, N_DEV)
        right = lax.rem(my + 1, N_DEV)

        barrier = pltpu.get_barrier_semaphore()
        pl.semaphore_signal(barrier, inc=1, device_id=(left,),
                            device_id_type=pl.DeviceIdType.MESH)
        pl.semaphore_signal(barrier, inc=1, device_id=(right,),
                            device_id_type=pl.DeviceIdType.MESH)
        pl.semaphore_wait(barrier, 2)

        def lc(c, col0):
            xc = x_ref[pl.ds(c * m_per, m_per), :]
            return lax.dot_general(
                xc, w_ref[:, col0:col0 + ns],
                dimension_numbers=(((1,), (0,)), ((), ())),
                preferred_element_type=jnp.float32,
            )

        def c_r(s):
            return lax.rem(my + N_DEV - s - 1, N_DEV)

        def c_l(s):
            return lax.rem(my + s + 1, N_DEV)

        def make(direction, s, b):
            if direction == 0:
                return pltpu.make_async_remote_copy(
                    src_ref=send_r.at[s, b],
                    dst_ref=recv_r.at[s, b],
                    send_sem=send_sems_r.at[s, b],
                    recv_sem=recv_sems_r.at[s, b],
                    device_id=(right,),
                    device_id_type=pl.DeviceIdType.MESH,
                )
            return pltpu.make_async_remote_copy(
                src_ref=send_l.at[s, b],
                dst_ref=recv_l.at[s, b],
                send_sem=send_sems_l.at[s, b],
                recv_sem=recv_sems_l.at[s, b],
                device_id=(left,),
                device_id_type=pl.DeviceIdType.MESH,
            )

        rdmas = {}

        for b in range(N_SUB):
            send_r[0, b, :, :] = lc(c_r(0), b * ns)
            rdmas[(0, 0, b)] = make(0, 0, b)
            rdmas[(0, 0, b)].start()
            send_l[0, b, :, :] = lc(c_l(0), nh + b * ns)
            rdmas[(1, 0, b)] = make(1, 0, b)
            rdmas[(1, 0, b)].start()

        for s in range(N_DEV - 1):
            last = s == N_DEV - 2
            for b in range(N_SUB):
                if last:
                    nxt_r = lc(my, b * ns)
                    nxt_l = lc(my, nh + b * ns)
                else:
                    nxt_r = lc(c_r(s + 1), b * ns)
                    nxt_l = lc(c_l(s + 1), nh + b * ns)

                rdmas[(0, s, b)].wait_recv()
                if last:
                    out_ref[:, b * ns:(b + 1) * ns] = jnp.maximum(
                        recv_r[s, b, :, :] + nxt_r, 0.0)
                else:
                    send_r[s + 1, b, :, :] = recv_r[s, b, :, :] + nxt_r
                    rdmas[(0, s + 1, b)] = make(0, s + 1, b)
                    rdmas[(0, s + 1, b)].start()

                rdmas[(1, s, b)].wait_recv()
                if last:
                    out_ref[:, nh + b * ns:nh + (b + 1) * ns] = jnp.maximum(
                        recv_l[s, b, :, :] + nxt_l, 0.0)
                else:
                    send_l[s + 1, b, :, :] = recv_l[s, b, :, :] + nxt_l
                    rdmas[(1, s + 1, b)] = make(1, s + 1, b)
                    rdmas[(1, s + 1, b)].start()

        for d in range(2):
            for s in range(N_DEV - 1):
                for b in range(N_SUB):
                    rdmas[(d, s, b)].wait_send()

    nslots = N_DEV - 1
    return pl.pallas_call(
        body,
        out_shape=jax.ShapeDtypeStruct((m_per, n), jnp.float32),
        in_specs=[
            pl.BlockSpec(memory_space=pltpu.VMEM),
            pl.BlockSpec(memory_space=pltpu.VMEM),
        ],
        out_specs=pl.BlockSpec(memory_space=pltpu.VMEM),
        scratch_shapes=[
            pltpu.VMEM((nslots, N_SUB, m_per, ns), jnp.float32),
            pltpu.VMEM((nslots, N_SUB, m_per, ns), jnp.float32),
            pltpu.VMEM((nslots, N_SUB, m_per, ns), jnp.float32),
            pltpu.VMEM((nslots, N_SUB, m_per, ns), jnp.float32),
            pltpu.SemaphoreType.DMA((nslots, N_SUB)),
            pltpu.SemaphoreType.DMA((nslots, N_SUB)),
            pltpu.SemaphoreType.DMA((nslots, N_SUB)),
            pltpu.SemaphoreType.DMA((nslots, N_SUB)),
        ],
        compiler_params=pltpu.CompilerParams(collective_id=0),
    )(x, w_mat)


# device time: 90097 ns/iter; 1.0053x vs baseline; 1.0053x over previous
import jax
import jax.numpy as jnp
from jax import lax
from jax.experimental import pallas as pl
from jax.experimental.pallas import tpu as pltpu

N_DEV = 8
N_SUB = 2


def kernel(x, w_mat):
    m_global, k_shard = x.shape
    _, n = w_mat.shape
    m_per = m_global // N_DEV
    nh = n // 2
    ns = nh // N_SUB

    def body(x_ref, w_ref, out_ref,
             send_r, send_l, recv_r, recv_l,
             send_sems_r, send_sems_l, recv_sems_r, recv_sems_l):
        my = lax.axis_index("i")
        left = lax.rem(my + N_DEV - 1, N_DEV)
        right = lax.rem(my + 1, N_DEV)

        barrier = pltpu.get_barrier_semaphore()
        pl.semaphore_signal(barrier, inc=1, device_id=(left,),
                            device_id_type=pl.DeviceIdType.MESH)
        pl.semaphore_signal(barrier, inc=1, device_id=(right,),
                            device_id_type=pl.DeviceIdType.MESH)
        pl.semaphore_wait(barrier, 2)

        def lc(c, col0):
            xc = x_ref[pl.ds(c * m_per, m_per), :].astype(jnp.bfloat16)
            wc = w_ref[:, col0:col0 + ns].astype(jnp.bfloat16)
            return lax.dot_general(
                xc, wc,
                dimension_numbers=(((1,), (0,)), ((), ())),
                preferred_element_type=jnp.float32,
            )

        def c_r(s):
            return lax.rem(my + N_DEV - s - 1, N_DEV)

        def c_l(s):
            return lax.rem(my + s + 1, N_DEV)

        def make(direction, s, b):
            if direction == 0:
                return pltpu.make_async_remote_copy(
                    src_ref=send_r.at[s, b],
                    dst_ref=recv_r.at[s, b],
                    send_sem=send_sems_r.at[s, b],
                    recv_sem=recv_sems_r.at[s, b],
                    device_id=(right,),
                    device_id_type=pl.DeviceIdType.MESH,
                )
            return pltpu.make_async_remote_copy(
                src_ref=send_l.at[s, b],
                dst_ref=recv_l.at[s, b],
                send_sem=send_sems_l.at[s, b],
                recv_sem=recv_sems_l.at[s, b],
                device_id=(left,),
                device_id_type=pl.DeviceIdType.MESH,
            )

        rdmas = {}

        for b in range(N_SUB):
            send_r[0, b, :, :] = lc(c_r(0), b * ns)
            rdmas[(0, 0, b)] = make(0, 0, b)
            rdmas[(0, 0, b)].start()
            send_l[0, b, :, :] = lc(c_l(0), nh + b * ns)
            rdmas[(1, 0, b)] = make(1, 0, b)
            rdmas[(1, 0, b)].start()

        for s in range(N_DEV - 1):
            last = s == N_DEV - 2
            for b in range(N_SUB):
                if last:
                    nxt_r = lc(my, b * ns)
                    nxt_l = lc(my, nh + b * ns)
                else:
                    nxt_r = lc(c_r(s + 1), b * ns)
                    nxt_l = lc(c_l(s + 1), nh + b * ns)

                rdmas[(0, s, b)].wait_recv()
                if last:
                    out_ref[:, b * ns:(b + 1) * ns] = jnp.maximum(
                        recv_r[s, b, :, :] + nxt_r, 0.0)
                else:
                    send_r[s + 1, b, :, :] = recv_r[s, b, :, :] + nxt_r
                    rdmas[(0, s + 1, b)] = make(0, s + 1, b)
                    rdmas[(0, s + 1, b)].start()

                rdmas[(1, s, b)].wait_recv()
                if last:
                    out_ref[:, nh + b * ns:nh + (b + 1) * ns] = jnp.maximum(
                        recv_l[s, b, :, :] + nxt_l, 0.0)
                else:
                    send_l[s + 1, b, :, :] = recv_l[s, b, :, :] + nxt_l
                    rdmas[(1, s + 1, b)] = make(1, s + 1, b)
                    rdmas[(1, s + 1, b)].start()

        for d in range(2):
            for s in range(N_DEV - 1):
                for b in range(N_SUB):
                    rdmas[(d, s, b)].wait_send()

    nslots = N_DEV - 1
    return pl.pallas_call(
        body,
        out_shape=jax.ShapeDtypeStruct((m_per, n), jnp.float32),
        in_specs=[
            pl.BlockSpec(memory_space=pltpu.VMEM),
            pl.BlockSpec(memory_space=pltpu.VMEM),
        ],
        out_specs=pl.BlockSpec(memory_space=pltpu.VMEM),
        scratch_shapes=[
            pltpu.VMEM((nslots, N_SUB, m_per, ns), jnp.float32),
            pltpu.VMEM((nslots, N_SUB, m_per, ns), jnp.float32),
            pltpu.VMEM((nslots, N_SUB, m_per, ns), jnp.float32),
            pltpu.VMEM((nslots, N_SUB, m_per, ns), jnp.float32),
            pltpu.SemaphoreType.DMA((nslots, N_SUB)),
            pltpu.SemaphoreType.DMA((nslots, N_SUB)),
            pltpu.SemaphoreType.DMA((nslots, N_SUB)),
            pltpu.SemaphoreType.DMA((nslots, N_SUB)),
        ],
        compiler_params=pltpu.CompilerParams(collective_id=0),
    )(x, w_mat)


# device time: 65303 ns/iter; 1.3870x vs baseline; 1.3797x over previous
import jax
import jax.numpy as jnp
from jax import lax
from jax.experimental import pallas as pl
from jax.experimental.pallas import tpu as pltpu

N_DEV = 8
PART_COLS = (768, 640, 640)
PART_OFF = (0, 768, 1408)
DIMS = ((0, 1, 2), (1, 2, 0), (2, 0, 1))


def kernel(x, w_mat):
    m_global, k_shard = x.shape
    _, n = w_mat.shape
    m_per = m_global // N_DEV

    def body(x_ref, w_ref, out_ref,
             acc0, acc1, acc2, rcv0, rcv1, rcv2,
             send_sems, recv_sems):
        accs = (acc0, acc1, acc2)
        rcvs = (rcv0, rcv1, rcv2)

        i = lax.axis_index("i")
        mz = i // 4
        j = lax.rem(i, 4)
        my_ = j // 2
        mx = my_ ^ lax.rem(j, 2)
        me = (mx, my_, mz)

        def ring(cx, cy, cz):
            return 4 * cz + 2 * cy + (cx ^ cy)

        def flipped(d):
            c = list(me)
            c[d] = 1 - c[d]
            return ring(*c)

        partner = [[flipped(DIMS[p][r]) for r in range(3)] for p in range(3)]

        barrier = pltpu.get_barrier_semaphore()
        for d in range(3):
            pl.semaphore_signal(barrier, inc=1, device_id=(flipped(d),),
                                device_id_type=pl.DeviceIdType.MESH)
        pl.semaphore_wait(barrier, 3)

        def chunk_of(p, t):
            bits = [(t >> 2) & 1, (t >> 1) & 1, t & 1]
            c = list(me)
            for k in range(3):
                if bits[k]:
                    c[DIMS[p][k]] = 1 - c[DIMS[p][k]]
            return ring(*c)

        def gemm(p, t):
            c = chunk_of(p, t)
            xc = x_ref[pl.ds(c * m_per, m_per), :]
            wc = w_ref[:, PART_OFF[p]:PART_OFF[p] + PART_COLS[p]]
            return lax.dot_general(
                xc, wc,
                dimension_numbers=(((1,), (0,)), ((), ())),
                preferred_element_type=jnp.float32,
            )

        def sem_idx(p, r, b):
            return p * 5 + (0, 2, 4)[r] + b

        def make(p, r, b):
            if r == 0:
                src = slice(6 - 2 * b, 8 - 2 * b)
                dst = slice(2 - 2 * b, 4 - 2 * b)
            elif r == 1:
                src = slice(3 - b, 4 - b)
                dst = slice(5 - b, 6 - b)
            else:
                src = slice(1, 2)
                dst = slice(6, 7)
            k = sem_idx(p, r, b)
            return pltpu.make_async_remote_copy(
                src_ref=accs[p].at[src],
                dst_ref=rcvs[p].at[dst],
                send_sem=send_sems.at[k],
                recv_sem=recv_sems.at[k],
                device_id=(partner[p][r],),
                device_id_type=pl.DeviceIdType.MESH,
            )

        rdmas = {}

        def launch(p, r, b):
            rd = make(p, r, b)
            rd.start()
            rdmas[(p, r, b)] = rd

        for p in range(3):
            for t in (6, 7):
                accs[p][t, :, :] = gemm(p, t)
            launch(p, 0, 0)
        for p in range(3):
            for t in (4, 5):
                accs[p][t, :, :] = gemm(p, t)
            launch(p, 0, 1)
        for p in range(3):
            for t in (2, 3, 0, 1):
                accs[p][t, :, :] = gemm(p, t)

        for p in range(3):
            rdmas[(p, 0, 0)].wait_recv()
            accs[p][2:4, :, :] = accs[p][2:4, :, :] + rcvs[p][2:4, :, :]
            launch(p, 1, 0)
            launch(p, 1, 1)
        for p in range(3):
            rdmas[(p, 0, 1)].wait_recv()
            accs[p][0:2, :, :] = accs[p][0:2, :, :] + rcvs[p][0:2, :, :]

        for p in range(3):
            rdmas[(p, 1, 0)].wait_recv()
            accs[p][1, :, :] = accs[p][1, :, :] + rcvs[p][5, :, :]
            launch(p, 2, 0)
        for p in range(3):
            rdmas[(p, 1, 1)].wait_recv()
            accs[p][0, :, :] = accs[p][0, :, :] + rcvs[p][4, :, :]

        for p in range(3):
            rdmas[(p, 2, 0)].wait_recv()
            out_ref[:, PART_OFF[p]:PART_OFF[p] + PART_COLS[p]] = jnp.maximum(
                accs[p][0, :, :] + rcvs[p][6, :, :], 0.0)

        for key in rdmas:
            rdmas[key].wait_send()

    return pl.pallas_call(
        body,
        out_shape=jax.ShapeDtypeStruct((m_per, n), jnp.float32),
        in_specs=[
            pl.BlockSpec(memory_space=pltpu.VMEM),
            pl.BlockSpec(memory_space=pltpu.VMEM),
        ],
        out_specs=pl.BlockSpec(memory_space=pltpu.VMEM),
        scratch_shapes=[
            pltpu.VMEM((N_DEV, m_per, PART_COLS[0]), jnp.float32),
            pltpu.VMEM((N_DEV, m_per, PART_COLS[1]), jnp.float32),
            pltpu.VMEM((N_DEV, m_per, PART_COLS[2]), jnp.float32),
            pltpu.VMEM((7, m_per, PART_COLS[0]), jnp.float32),
            pltpu.VMEM((7, m_per, PART_COLS[1]), jnp.float32),
            pltpu.VMEM((7, m_per, PART_COLS[2]), jnp.float32),
            pltpu.SemaphoreType.DMA((15,)),
            pltpu.SemaphoreType.DMA((15,)),
        ],
        compiler_params=pltpu.CompilerParams(collective_id=0),
    )(x, w_mat)
